# baseline (speedup 1.0000x reference)
import functools

import jax
import jax.numpy as jnp
from jax import lax
from jax.experimental import pallas as pl
from jax.experimental.pallas import tpu as pltpu

_ALPHA = 32
_RANK = 16


def _nt_dot(a, b):
    return lax.dot_general(a, b, (((1,), (1,)), ((), ())),
                           preferred_element_type=jnp.float32)


def _nn_dot(a, b):
    return lax.dot_general(a, b, (((1,), (0,)), ((), ())),
                           preferred_element_type=jnp.float32)


def _expand_routing(routing, er):
    n_e = routing.shape[1]
    r0 = lax.broadcasted_iota(jnp.int32, (n_e, er), 0)
    r1 = lax.broadcasted_iota(jnp.int32, (n_e, er), 1)
    expand = (r1 // _RANK == r0).astype(jnp.float32)
    return _nn_dot(routing, expand)


def _router_body(x_ref, w_ref, b_ref, rout_ref, ec_ref, xbf_ref):
    x = x_ref[...]
    w = w_ref[...]
    logits = _nt_dot(x, w) + b_ref[0:1, :]
    mx = jnp.max(logits, axis=-1, keepdims=True)
    e = jnp.exp(logits - mx)
    routing = e / jnp.sum(e, axis=-1, keepdims=True)
    rmax = jnp.max(routing, axis=-1, keepdims=True)
    lane = lax.broadcasted_iota(jnp.int32, routing.shape, 1)
    first = jnp.min(jnp.where(routing == rmax, lane, routing.shape[-1]),
                    axis=-1, keepdims=True)
    y_hard = (lane == first).astype(jnp.float32)
    rout_ref[...] = routing
    ec_ref[...] = (y_hard - routing) + routing
    xbf_ref[...] = x.astype(jnp.bfloat16)


def _gateup_body(scaling, nm,
                 xbf_ref, rout_ref, gw_ref, uw_ref, dwf_ref, adf_ref,
                 bg_ref, bu_ref, ag_ref, au_ref,
                 h_ref, dwb_ref, ha_ref, hwg_ref, hwu_ref, had_ref):
    m = pl.program_id(1)
    er = ag_ref.shape[0]
    mt = gw_ref.shape[0]
    mh = mt // 2

    @pl.when(m == 0)
    def _init():
        rrep = _expand_routing(rout_ref[...], er)
        xbf = xbf_ref[...]
        xag = _nt_dot(xbf, ag_ref[...].astype(jnp.bfloat16))
        xau = _nt_dot(xbf, au_ref[...].astype(jnp.bfloat16))
        hwg_ref[...] = (xag * rrep * scaling).astype(jnp.bfloat16)
        hwu_ref[...] = (xau * rrep * scaling).astype(jnp.bfloat16)

    xbf = xbf_ref[...]
    hwg = hwg_ref[...]
    hwu = hwu_ref[...]
    ha = jnp.zeros((xbf.shape[0], er), jnp.float32)
    for k in range(2):
        sl = slice(k * mh, (k + 1) * mh)
        g = (_nt_dot(xbf, gw_ref[sl, :].astype(jnp.bfloat16))
             + _nn_dot(hwg, bg_ref[:, sl].astype(jnp.bfloat16)))
        u = (_nt_dot(xbf, uw_ref[sl, :].astype(jnp.bfloat16))
             + _nn_dot(hwu, bu_ref[:, sl].astype(jnp.bfloat16)))
        hk = ((g * jax.nn.sigmoid(g)) * u).astype(jnp.bfloat16)
        h_ref[:, sl] = hk
        ha = ha + _nt_dot(hk, adf_ref[:, sl].astype(jnp.bfloat16))
    dwb_ref[...] = dwf_ref[...].astype(jnp.bfloat16)

    @pl.when(m == 0)
    def _ha_first():
        had_ref[...] = ha

    @pl.when(m > 0)
    def _ha_acc():
        had_ref[...] += ha

    @pl.when(m == nm - 1)
    def _ha_out():
        ha_ref[...] = had_ref[...]


def _down_body(scaling, h_ref, rout_ref, dw_ref, ha_ref, bd_ref, out_ref):
    er = bd_ref.shape[0]
    d = dw_ref.shape[0]
    dh = d // 4
    h = h_ref[...]
    rrep = _expand_routing(rout_ref[...], er)
    hwd = (ha_ref[...] * rrep * scaling).astype(jnp.bfloat16)
    for k in range(4):
        sl = slice(k * dh, (k + 1) * dh)
        out_ref[:, sl] = (_nt_dot(h, dw_ref[sl, :])
                          + _nn_dot(hwd, bd_ref[:, sl].astype(jnp.bfloat16)))


def _pipeline(x, gate_W, up_W, down_W, router_W, router_b,
              gate_A, gate_B, up_A, up_B, down_A, down_B):
    b, s, d = x.shape
    m_dim = gate_W.shape[0]
    n_e, rank, _ = gate_A.shape
    er = n_e * rank
    n = b * s
    scaling = _ALPHA / _RANK

    x2 = x.reshape(n, d)
    t_r = min(2048, n)
    nt_r = n // t_r
    routing, ec, xbf = pl.pallas_call(
        _router_body,
        grid=(nt_r,),
        in_specs=[
            pl.BlockSpec((t_r, d), lambda t: (t, 0)),
            pl.BlockSpec((n_e, d), lambda t: (0, 0)),
            pl.BlockSpec((8, n_e), lambda t: (0, 0)),
        ],
        out_specs=[
            pl.BlockSpec((t_r, n_e), lambda t: (t, 0)),
            pl.BlockSpec((t_r, n_e), lambda t: (t, 0)),
            pl.BlockSpec((t_r, d), lambda t: (t, 0)),
        ],
        out_shape=[
            jax.ShapeDtypeStruct((n, n_e), jnp.float32),
            jax.ShapeDtypeStruct((n, n_e), jnp.float32),
            jax.ShapeDtypeStruct((n, d), jnp.bfloat16),
        ],
    )(x2, router_W, jnp.broadcast_to(router_b.reshape(1, n_e), (8, n_e)))

    bg = gate_B.transpose(0, 2, 1).reshape(er, m_dim)
    bu = up_B.transpose(0, 2, 1).reshape(er, m_dim)
    adf = down_A.reshape(er, m_dim)
    ag = gate_A.reshape(er, d)
    au = up_A.reshape(er, d)
    bd = down_B.transpose(0, 2, 1).reshape(er, d)

    t = min(1024, n)
    mt = min(512, m_dim)
    nt = n // t
    nm = m_dim // mt
    h, dwb, ha = pl.pallas_call(
        functools.partial(_gateup_body, scaling, m_dim // mt),
        grid=(nt, nm),
        in_specs=[
            pl.BlockSpec((t, d), lambda i, j: (i, 0)),
            pl.BlockSpec((t, n_e), lambda i, j: (i, 0)),
            pl.BlockSpec((mt, d), lambda i, j: (j, 0)),
            pl.BlockSpec((mt, d), lambda i, j: (j, 0)),
            pl.BlockSpec((d, mt), lambda i, j: (0, j)),
            pl.BlockSpec((er, mt), lambda i, j: (0, j)),
            pl.BlockSpec((er, mt), lambda i, j: (0, j)),
            pl.BlockSpec((er, mt), lambda i, j: (0, j)),
            pl.BlockSpec((er, d), lambda i, j: (0, 0)),
            pl.BlockSpec((er, d), lambda i, j: (0, 0)),
        ],
        out_specs=[
            pl.BlockSpec((t, mt), lambda i, j: (i, j)),
            pl.BlockSpec((d, mt), lambda i, j: (0, j)),
            pl.BlockSpec((t, er), lambda i, j: (i, 0)),
        ],
        out_shape=[
            jax.ShapeDtypeStruct((n, m_dim), jnp.bfloat16),
            jax.ShapeDtypeStruct((d, m_dim), jnp.bfloat16),
            jax.ShapeDtypeStruct((n, er), jnp.float32),
        ],
        scratch_shapes=[
            pltpu.VMEM((t, er), jnp.bfloat16),
            pltpu.VMEM((t, er), jnp.bfloat16),
            pltpu.VMEM((t, er), jnp.float32),
        ],
        compiler_params=pltpu.CompilerParams(
            dimension_semantics=("parallel", "arbitrary"),
        ),
    )(xbf, routing, gate_W, up_W, down_W, adf, bg, bu, ag, au)

    t2 = min(256, n)
    nt2 = n // t2
    out = pl.pallas_call(
        functools.partial(_down_body, scaling),
        grid=(nt2,),
        in_specs=[
            pl.BlockSpec((t2, m_dim), lambda i: (i, 0)),
            pl.BlockSpec((t2, n_e), lambda i: (i, 0)),
            pl.BlockSpec((d, m_dim), lambda i: (0, 0)),
            pl.BlockSpec((t2, er), lambda i: (i, 0)),
            pl.BlockSpec((er, d), lambda i: (0, 0)),
        ],
        out_specs=pl.BlockSpec((t2, d), lambda i: (i, 0)),
        out_shape=jax.ShapeDtypeStruct((n, d), jnp.float32),
        compiler_params=pltpu.CompilerParams(
            dimension_semantics=("parallel",),
        ),
    )(h, routing, dwb, ha, bd)

    return (out.reshape(b, s, d),
            routing.reshape(b, s, n_e),
            ec.reshape(b, s, n_e))


def kernel(x, gate_W, up_W, down_W, router_W, router_b,
           gate_A, gate_B, up_A, up_B, down_A, down_B):
    return _pipeline(x, gate_W, up_W, down_W, router_W, router_b,
                     gate_A, gate_B, up_A, up_B, down_A, down_B)


# speedup vs baseline: 1.0020x; 1.0020x over previous
import functools

import jax
import jax.numpy as jnp
from jax import lax
from jax.experimental import pallas as pl
from jax.experimental.pallas import tpu as pltpu

_ALPHA = 32
_RANK = 16


def _nt_dot(a, b):
    return lax.dot_general(a, b, (((1,), (1,)), ((), ())),
                           preferred_element_type=jnp.float32)


def _nn_dot(a, b):
    return lax.dot_general(a, b, (((1,), (0,)), ((), ())),
                           preferred_element_type=jnp.float32)


def _expand_routing(routing, er):
    n_e = routing.shape[1]
    r0 = lax.broadcasted_iota(jnp.int32, (n_e, er), 0)
    r1 = lax.broadcasted_iota(jnp.int32, (n_e, er), 1)
    expand = (r1 // _RANK == r0).astype(jnp.float32)
    return _nn_dot(routing, expand)


def _router_body(x_ref, w_ref, b_ref, rout_ref, ec_ref, xbf_ref):
    x = x_ref[...]
    w = w_ref[...]
    logits = _nt_dot(x, w) + b_ref[0:1, :]
    mx = jnp.max(logits, axis=-1, keepdims=True)
    e = jnp.exp(logits - mx)
    routing = e / jnp.sum(e, axis=-1, keepdims=True)
    rmax = jnp.max(routing, axis=-1, keepdims=True)
    lane = lax.broadcasted_iota(jnp.int32, routing.shape, 1)
    first = jnp.min(jnp.where(routing == rmax, lane, routing.shape[-1]),
                    axis=-1, keepdims=True)
    y_hard = (lane == first).astype(jnp.float32)
    rout_ref[...] = routing
    ec_ref[...] = (y_hard - routing) + routing
    xbf_ref[...] = x.astype(jnp.bfloat16)


def _gateup_body(scaling, nm,
                 xbf_ref, rout_ref, gw_ref, uw_ref, dwf_ref, adf_ref,
                 bg_ref, bu_ref, ag_ref, au_ref,
                 h_ref, dwb_ref, ha_ref, hwg_ref, hwu_ref, had_ref):
    m = pl.program_id(1)
    er = ag_ref.shape[0]
    mt = gw_ref.shape[0]
    mh = mt // 2

    @pl.when(m == 0)
    def _init():
        rrep = _expand_routing(rout_ref[...], er)
        xbf = xbf_ref[...]
        xag = _nt_dot(xbf, ag_ref[...].astype(jnp.bfloat16))
        xau = _nt_dot(xbf, au_ref[...].astype(jnp.bfloat16))
        hwg_ref[...] = (xag * rrep * scaling).astype(jnp.bfloat16)
        hwu_ref[...] = (xau * rrep * scaling).astype(jnp.bfloat16)

    xbf = xbf_ref[...]
    hwg = hwg_ref[...]
    hwu = hwu_ref[...]
    hks = []
    ha = jnp.zeros((xbf.shape[0], er), jnp.float32)
    for k in range(2):
        sl = slice(k * mh, (k + 1) * mh)
        g = (_nt_dot(xbf, gw_ref[sl, :].astype(jnp.bfloat16))
             + _nn_dot(hwg, bg_ref[:, sl].astype(jnp.bfloat16)))
        u = (_nt_dot(xbf, uw_ref[sl, :].astype(jnp.bfloat16))
             + _nn_dot(hwu, bu_ref[:, sl].astype(jnp.bfloat16)))
        hk = ((g * jax.nn.sigmoid(g)) * u).astype(jnp.bfloat16)
        hks.append(hk)
        ha = ha + _nt_dot(hk, adf_ref[:, sl].astype(jnp.bfloat16))
    h_ref[...] = jnp.concatenate(hks, axis=1)
    dwb_ref[...] = dwf_ref[...].astype(jnp.bfloat16)

    @pl.when(m == 0)
    def _ha_first():
        had_ref[...] = ha

    @pl.when(m > 0)
    def _ha_acc():
        had_ref[...] += ha

    @pl.when(m == nm - 1)
    def _ha_out():
        ha_ref[...] = had_ref[...]


def _down_body(scaling, h_ref, rout_ref, dw_ref, ha_ref, bd_ref, out_ref):
    er = bd_ref.shape[0]
    d = dw_ref.shape[0]
    dh = d // 4
    h = h_ref[...]
    rrep = _expand_routing(rout_ref[...], er)
    hwd = (ha_ref[...] * rrep * scaling).astype(jnp.bfloat16)
    outs = []
    for k in range(4):
        sl = slice(k * dh, (k + 1) * dh)
        outs.append(_nt_dot(h, dw_ref[sl, :])
                    + _nn_dot(hwd, bd_ref[:, sl].astype(jnp.bfloat16)))
    out_ref[...] = jnp.concatenate(outs, axis=1)


def _pipeline(x, gate_W, up_W, down_W, router_W, router_b,
              gate_A, gate_B, up_A, up_B, down_A, down_B):
    b, s, d = x.shape
    m_dim = gate_W.shape[0]
    n_e, rank, _ = gate_A.shape
    er = n_e * rank
    n = b * s
    scaling = _ALPHA / _RANK

    x2 = x.reshape(n, d)
    t_r = min(2048, n)
    nt_r = n // t_r
    routing, ec, xbf = pl.pallas_call(
        _router_body,
        grid=(nt_r,),
        in_specs=[
            pl.BlockSpec((t_r, d), lambda t: (t, 0)),
            pl.BlockSpec((n_e, d), lambda t: (0, 0)),
            pl.BlockSpec((8, n_e), lambda t: (0, 0)),
        ],
        out_specs=[
            pl.BlockSpec((t_r, n_e), lambda t: (t, 0)),
            pl.BlockSpec((t_r, n_e), lambda t: (t, 0)),
            pl.BlockSpec((t_r, d), lambda t: (t, 0)),
        ],
        out_shape=[
            jax.ShapeDtypeStruct((n, n_e), jnp.float32),
            jax.ShapeDtypeStruct((n, n_e), jnp.float32),
            jax.ShapeDtypeStruct((n, d), jnp.bfloat16),
        ],
    )(x2, router_W, jnp.broadcast_to(router_b.reshape(1, n_e), (8, n_e)))

    bg = gate_B.transpose(0, 2, 1).reshape(er, m_dim)
    bu = up_B.transpose(0, 2, 1).reshape(er, m_dim)
    adf = down_A.reshape(er, m_dim)
    ag = gate_A.reshape(er, d)
    au = up_A.reshape(er, d)
    bd = down_B.transpose(0, 2, 1).reshape(er, d)

    t = min(1024, n)
    mt = min(512, m_dim)
    nt = n // t
    nm = m_dim // mt
    h, dwb, ha = pl.pallas_call(
        functools.partial(_gateup_body, scaling, m_dim // mt),
        grid=(nt, nm),
        in_specs=[
            pl.BlockSpec((t, d), lambda i, j: (i, 0)),
            pl.BlockSpec((t, n_e), lambda i, j: (i, 0)),
            pl.BlockSpec((mt, d), lambda i, j: (j, 0)),
            pl.BlockSpec((mt, d), lambda i, j: (j, 0)),
            pl.BlockSpec((d, mt), lambda i, j: (0, j)),
            pl.BlockSpec((er, mt), lambda i, j: (0, j)),
            pl.BlockSpec((er, mt), lambda i, j: (0, j)),
            pl.BlockSpec((er, mt), lambda i, j: (0, j)),
            pl.BlockSpec((er, d), lambda i, j: (0, 0)),
            pl.BlockSpec((er, d), lambda i, j: (0, 0)),
        ],
        out_specs=[
            pl.BlockSpec((t, mt), lambda i, j: (i, j)),
            pl.BlockSpec((d, mt), lambda i, j: (0, j)),
            pl.BlockSpec((t, er), lambda i, j: (i, 0)),
        ],
        out_shape=[
            jax.ShapeDtypeStruct((n, m_dim), jnp.bfloat16),
            jax.ShapeDtypeStruct((d, m_dim), jnp.bfloat16),
            jax.ShapeDtypeStruct((n, er), jnp.float32),
        ],
        scratch_shapes=[
            pltpu.VMEM((t, er), jnp.bfloat16),
            pltpu.VMEM((t, er), jnp.bfloat16),
            pltpu.VMEM((t, er), jnp.float32),
        ],
        compiler_params=pltpu.CompilerParams(
            dimension_semantics=("parallel", "arbitrary"),
        ),
    )(xbf, routing, gate_W, up_W, down_W, adf, bg, bu, ag, au)

    t2 = min(256, n)
    nt2 = n // t2
    out = pl.pallas_call(
        functools.partial(_down_body, scaling),
        grid=(nt2,),
        in_specs=[
            pl.BlockSpec((t2, m_dim), lambda i: (i, 0)),
            pl.BlockSpec((t2, n_e), lambda i: (i, 0)),
            pl.BlockSpec((d, m_dim), lambda i: (0, 0)),
            pl.BlockSpec((t2, er), lambda i: (i, 0)),
            pl.BlockSpec((er, d), lambda i: (0, 0)),
        ],
        out_specs=pl.BlockSpec((t2, d), lambda i: (i, 0)),
        out_shape=jax.ShapeDtypeStruct((n, d), jnp.float32),
        compiler_params=pltpu.CompilerParams(
            dimension_semantics=("parallel",),
        ),
    )(h, routing, dwb, ha, bd)

    return (out.reshape(b, s, d),
            routing.reshape(b, s, n_e),
            ec.reshape(b, s, n_e))


def kernel(x, gate_W, up_W, down_W, router_W, router_b,
           gate_A, gate_B, up_A, up_B, down_A, down_B):
    return _pipeline(x, gate_W, up_W, down_W, router_W, router_b,
                     gate_A, gate_B, up_A, up_B, down_A, down_B)
